# Initial kernel scaffold; baseline (speedup 1.0000x reference)
import jax
import jax.numpy as jnp
from jax.experimental import pallas as pl


def kernel(inputs, emb, W, b):
    raise NotImplementedError("write your pallas kernel here")


# speedup vs baseline: 2.6044x; 2.6044x over previous
import functools
import jax
import jax.numpy as jnp
from jax.experimental import pallas as pl


VOCAB_PAD = 1024


def _m_kernel(emb_ref, w_ref, m_ref, *, inv_l):
    m_ref[...] = jax.lax.dot_general(
        emb_ref[...], w_ref[...],
        dimension_numbers=(((1,), (1,)), ((), ())),
        preferred_element_type=jnp.float32,
    ) * inv_l


def _fused_kernel(idx_ref, m_ref, b_ref, out_ref, *, hist_len):
    tb = idx_ref.shape[1]

    iota_v = jax.lax.broadcasted_iota(jnp.int32, (VOCAB_PAD, tb), 0)

    def body(l, counts):
        row = idx_ref[pl.ds(l, 1), :]
        return counts + jnp.where(row == iota_v, 1.0, 0.0)

    counts_t = jax.lax.fori_loop(
        0, hist_len, body, jnp.zeros((VOCAB_PAD, tb), jnp.float32))

    z = jax.lax.dot_general(
        counts_t, m_ref[...],
        dimension_numbers=(((0,), (0,)), ((), ())),
        preferred_element_type=jnp.float32,
    ) + b_ref[...]

    zmax = jnp.max(z, axis=1, keepdims=True)
    s = z - zmax
    lse = jnp.log(jnp.sum(jnp.exp(s), axis=1, keepdims=True))
    out_ref[...] = s - lse


def kernel(inputs, emb, W, b):
    B, L = inputs.shape
    V, D = emb.shape
    C = W.shape[0]

    emb_p = jnp.zeros((VOCAB_PAD, D), emb.dtype).at[:V].set(emb)

    m = pl.pallas_call(
        functools.partial(_m_kernel, inv_l=1.0 / L),
        out_shape=jax.ShapeDtypeStruct((VOCAB_PAD, C), jnp.float32),
    )(emb_p, W)

    TB = 512
    idx_t = inputs.T
    b2 = b.reshape(1, C)

    out = pl.pallas_call(
        functools.partial(_fused_kernel, hist_len=L),
        grid=(B // TB,),
        in_specs=[
            pl.BlockSpec((L, TB), lambda i: (0, i)),
            pl.BlockSpec((VOCAB_PAD, C), lambda i: (0, 0)),
            pl.BlockSpec((1, C), lambda i: (0, 0)),
        ],
        out_specs=pl.BlockSpec((TB, C), lambda i: (i, 0)),
        out_shape=jax.ShapeDtypeStruct((B, C), jnp.float32),
    )(idx_t, m, b2)

    return out


# speedup vs baseline: 30.9258x; 11.8744x over previous
import functools
import jax
import jax.numpy as jnp
from jax import lax
from jax.experimental import pallas as pl
from jax.experimental.pallas import tpu as pltpu
from jax.experimental.pallas import tpu_sc as plsc


VOCAB_PAD = 1024
SENTINEL = VOCAB_PAD - 1
LANES = 16
NC, NS = 2, 16
NW = NC * NS
CHUNK = 16


def _m_kernel(emb_ref, w_ref, m_ref, *, inv_l):
    m_ref[...] = lax.dot_general(
        emb_ref[...], w_ref[...],
        dimension_numbers=(((1,), (1,)), ((), ())),
        preferred_element_type=jnp.float32,
    ) * inv_l


def _final_kernel(counts_ref, m_ref, b_ref, out_ref):
    z = lax.dot_general(
        counts_ref[...], m_ref[...],
        dimension_numbers=(((1,), (0,)), ((), ())),
        preferred_element_type=jnp.float32,
    ) + b_ref[...]
    zmax = jnp.max(z, axis=1, keepdims=True)
    s = z - zmax
    lse = jnp.log(jnp.sum(jnp.exp(s), axis=1, keepdims=True))
    out_ref[...] = s - lse


def _sc_hist_body(idx_hbm, zeros_hbm, out_hbm, idx_v, cnt0, cnt1, sem0, sem1,
                  *, rows_per_w, lp):
    wid = lax.axis_index("s") * NC + lax.axis_index("c")
    base = wid * rows_per_w

    pltpu.sync_copy(idx_hbm.at[pl.ds(base * lp, rows_per_w * lp)], idx_v)
    pltpu.sync_copy(zeros_hbm, cnt0)
    pltpu.sync_copy(zeros_hbm, cnt1)

    bufs = (cnt0, cnt1)
    sems = (sem0, sem1)
    nchunks = rows_per_w // CHUNK
    njv = lp // LANES
    ones = jnp.full((LANES,), 1.0, jnp.float32)
    zeros16 = jnp.zeros((LANES,), jnp.float32)

    def scatter_rows(cnt, chunk_id, values):
        def row_body(r, _):
            row_off = (chunk_id * CHUNK + r) * lp
            dst_off = r * VOCAB_PAD
            for j in range(njv):
                vidx = idx_v[pl.ds(row_off + j * LANES, LANES)] + dst_off
                if values is None:
                    plsc.store_scatter(cnt, [vidx], zeros16)
                else:
                    plsc.addupdate_scatter(cnt, [vidx], values)
            return 0
        lax.fori_loop(0, CHUNK, row_body, 0)

    pending = [None, None]
    for c in range(nchunks):
        buf = c % 2
        if c >= 2:
            pending[buf].wait()
            scatter_rows(bufs[buf], c - 2, None)
        scatter_rows(bufs[buf], c, ones)
        dst = out_hbm.at[pl.ds((base + c * CHUNK) * VOCAB_PAD,
                               CHUNK * VOCAB_PAD)]
        pending[buf] = pltpu.async_copy(bufs[buf], dst, sems[buf])
    pending[(nchunks - 2) % 2].wait()
    pending[(nchunks - 1) % 2].wait()


def _sc_hist(inputs_p, zeros, *, b, lp):
    rows_per_w = b // NW
    mesh = plsc.VectorSubcoreMesh(core_axis_name="c", subcore_axis_name="s")
    body = functools.partial(_sc_hist_body, rows_per_w=rows_per_w, lp=lp)
    f = pl.kernel(
        body,
        out_type=jax.ShapeDtypeStruct((b * VOCAB_PAD,), jnp.float32),
        mesh=mesh,
        scratch_types=[
            pltpu.VMEM((rows_per_w * lp,), jnp.int32),
            pltpu.VMEM((CHUNK * VOCAB_PAD,), jnp.float32),
            pltpu.VMEM((CHUNK * VOCAB_PAD,), jnp.float32),
            pltpu.SemaphoreType.DMA,
            pltpu.SemaphoreType.DMA,
        ],
        compiler_params=pltpu.CompilerParams(needs_layout_passes=False),
    )
    return f(inputs_p, zeros)


def kernel(inputs, emb, W, b):
    B, L = inputs.shape
    V, D = emb.shape
    C = W.shape[0]

    lp = ((L + LANES - 1) // LANES) * LANES
    pad = jnp.full((B, lp - L), SENTINEL, jnp.int32)
    inputs_p = jnp.concatenate([inputs, pad], axis=1).reshape(-1)

    zeros = jnp.zeros((CHUNK * VOCAB_PAD,), jnp.float32)
    counts = _sc_hist(inputs_p, zeros, b=B, lp=lp).reshape(B, VOCAB_PAD)

    emb_p = jnp.zeros((VOCAB_PAD, D), emb.dtype).at[:V].set(emb)
    m = pl.pallas_call(
        functools.partial(_m_kernel, inv_l=1.0 / L),
        out_shape=jax.ShapeDtypeStruct((VOCAB_PAD, C), jnp.float32),
    )(emb_p, W)

    TB = 512
    b2 = b.reshape(1, C)
    out = pl.pallas_call(
        _final_kernel,
        grid=(B // TB,),
        in_specs=[
            pl.BlockSpec((TB, VOCAB_PAD), lambda i: (i, 0)),
            pl.BlockSpec((VOCAB_PAD, C), lambda i: (0, 0)),
            pl.BlockSpec((1, C), lambda i: (0, 0)),
        ],
        out_specs=pl.BlockSpec((TB, C), lambda i: (i, 0)),
        out_shape=jax.ShapeDtypeStruct((B, C), jnp.float32),
    )(counts, m, b2)

    return out


# speedup vs baseline: 43.1868x; 1.3965x over previous
import functools
import jax
import jax.numpy as jnp
from jax import lax
from jax.experimental import pallas as pl
from jax.experimental.pallas import tpu as pltpu
from jax.experimental.pallas import tpu_sc as plsc


VOCAB_PAD = 1024
WORDS = VOCAB_PAD // 4
LANES = 16
NC, NS = 2, 16
NW = NC * NS


def _sc_hist_body(idx_hbm, out_hbm, idx_v, cnt_v, *, rows_per_w, hist):
    wid = lax.axis_index("s") * NC + lax.axis_index("c")
    base = wid * rows_per_w

    pltpu.sync_copy(idx_hbm.at[pl.ds(base, rows_per_w)], idx_v)

    zero16 = jnp.zeros((LANES,), jnp.int32)
    nzero = rows_per_w * WORDS // LANES

    def zbody(i, _):
        cnt_v[pl.ds(i * LANES, LANES)] = zero16
        return 0
    lax.fori_loop(0, nzero, zbody, 0)

    nfull = hist // LANES
    tail = hist - nfull * LANES
    one = jnp.full((LANES,), 1, jnp.int32)
    lane = lax.iota(jnp.int32, LANES)

    def scatter_group(dst_off, vidx, mask):
        word = (vidx & 255) + dst_off
        val = one << ((vidx >> 8) << 3)
        plsc.addupdate_scatter(cnt_v, [word], val, mask=mask)

    def row_body(r, _):
        dst_off = r * WORDS
        for j in range(nfull):
            vidx = idx_v[r, pl.ds(j * LANES, LANES)]
            scatter_group(dst_off, vidx, None)
        if tail:
            vidx = idx_v[r, pl.ds(hist - LANES, LANES)]
            scatter_group(dst_off, vidx, lane >= (LANES - tail))
        return 0
    lax.fori_loop(0, rows_per_w, row_body, 0)

    pltpu.sync_copy(cnt_v, out_hbm.at[pl.ds(base * WORDS, rows_per_w * WORDS)])


def _sc_hist(inputs, *, b, hist):
    rows_per_w = b // NW
    mesh = plsc.VectorSubcoreMesh(core_axis_name="c", subcore_axis_name="s")
    body = functools.partial(_sc_hist_body, rows_per_w=rows_per_w, hist=hist)
    f = pl.kernel(
        body,
        out_type=jax.ShapeDtypeStruct((b * WORDS,), jnp.int32),
        mesh=mesh,
        scratch_types=[
            pltpu.VMEM((rows_per_w, hist), jnp.int32),
            pltpu.VMEM((rows_per_w * WORDS,), jnp.int32),
        ],
        compiler_params=pltpu.CompilerParams(needs_layout_passes=False),
    )
    return f(inputs)


def _final_kernel(packed_ref, emb_ref, w_ref, b_ref, out_ref, m_ref, *,
                  inv_l, vocab):
    m_ref[...] = jnp.zeros_like(m_ref)
    m_ref[:vocab, :] = lax.dot_general(
        emb_ref[...], w_ref[...],
        dimension_numbers=(((1,), (1,)), ((), ())),
        preferred_element_type=jnp.float32,
    ) * inv_l

    packed = packed_ref[...]
    tb = packed.shape[0]
    z = jnp.zeros((tb, b_ref.shape[1]), jnp.float32) + b_ref[...]
    for p in range(4):
        plane = ((packed >> (8 * p)) & 255).astype(jnp.float32)
        z = z + lax.dot_general(
            plane, m_ref[pl.ds(256 * p, 256), :],
            dimension_numbers=(((1,), (0,)), ((), ())),
            preferred_element_type=jnp.float32,
        )
    zmax = jnp.max(z, axis=1, keepdims=True)
    s = z - zmax
    lse = jnp.log(jnp.sum(jnp.exp(s), axis=1, keepdims=True))
    out_ref[...] = s - lse


def kernel(inputs, emb, W, b):
    B, L = inputs.shape
    V, D = emb.shape
    C = W.shape[0]

    packed = _sc_hist(inputs, b=B, hist=L).reshape(B, WORDS)

    TB = 512
    b2 = b.reshape(1, C)
    out = pl.pallas_call(
        functools.partial(_final_kernel, inv_l=1.0 / L, vocab=V),
        grid=(B // TB,),
        in_specs=[
            pl.BlockSpec((TB, WORDS), lambda i: (i, 0)),
            pl.BlockSpec((V, D), lambda i: (0, 0)),
            pl.BlockSpec((C, D), lambda i: (0, 0)),
            pl.BlockSpec((1, C), lambda i: (0, 0)),
        ],
        out_specs=pl.BlockSpec((TB, C), lambda i: (i, 0)),
        out_shape=jax.ShapeDtypeStruct((B, C), jnp.float32),
        scratch_shapes=[pltpu.VMEM((VOCAB_PAD, C), jnp.float32)],
    )(packed, emb, W, b2)

    return out


# speedup vs baseline: 59.5688x; 1.3793x over previous
import functools
import jax
import jax.numpy as jnp
from jax import lax
from jax.experimental import pallas as pl
from jax.experimental.pallas import tpu as pltpu
from jax.experimental.pallas import tpu_sc as plsc


VOCAB_PAD = 1024
WORDS = VOCAB_PAD // 4
LANES = 16
NC, NS = 2, 16
NW = NC * NS


def _sc_hist_body(idx_hbm, out_hbm, idx_v, cnt_v, sem, *, rows_per_w, hist):
    wid = lax.axis_index("s") * NC + lax.axis_index("c")
    base = wid * rows_per_w

    stage = pltpu.async_copy(idx_hbm.at[pl.ds(base, rows_per_w)], idx_v, sem)

    zero16 = jnp.zeros((LANES,), jnp.int32)
    nzero = rows_per_w * WORDS // LANES

    @plsc.parallel_loop(0, nzero, unroll=8)
    def _(i):
        cnt_v[pl.ds(i * LANES, LANES)] = zero16

    stage.wait()

    nfull = hist // LANES
    tail = hist - nfull * LANES
    one = jnp.full((LANES,), 1, jnp.int32)
    lane = lax.iota(jnp.int32, LANES)

    def scatter_group(dst_off, vidx, mask):
        word = (vidx & 255) + dst_off
        val = one << ((vidx >> 8) << 3)
        plsc.addupdate_scatter(cnt_v, [word], val, mask=mask)

    @plsc.parallel_loop(0, rows_per_w, unroll=2)
    def _(r):
        dst_off = r * WORDS
        for j in range(nfull):
            vidx = idx_v[r, pl.ds(j * LANES, LANES)]
            scatter_group(dst_off, vidx, None)
        if tail:
            vidx = idx_v[r, pl.ds(hist - LANES, LANES)]
            scatter_group(dst_off, vidx, lane >= (LANES - tail))

    pltpu.sync_copy(cnt_v, out_hbm.at[pl.ds(base * WORDS, rows_per_w * WORDS)])


def _sc_hist(inputs, *, b, hist):
    rows_per_w = b // NW
    mesh = plsc.VectorSubcoreMesh(core_axis_name="c", subcore_axis_name="s")
    body = functools.partial(_sc_hist_body, rows_per_w=rows_per_w, hist=hist)
    f = pl.kernel(
        body,
        out_type=jax.ShapeDtypeStruct((b * WORDS,), jnp.int32),
        mesh=mesh,
        scratch_types=[
            pltpu.VMEM((rows_per_w, hist), jnp.int32),
            pltpu.VMEM((rows_per_w * WORDS,), jnp.int32),
            pltpu.SemaphoreType.DMA,
        ],
        compiler_params=pltpu.CompilerParams(needs_layout_passes=False),
    )
    return f(inputs)


def _final_kernel(packed_ref, emb_ref, w_ref, b_ref, out_ref, m_ref, *,
                  inv_l, vocab):
    m_ref[...] = jnp.zeros_like(m_ref)
    m_ref[:vocab, :] = lax.dot_general(
        emb_ref[...], w_ref[...],
        dimension_numbers=(((1,), (1,)), ((), ())),
        preferred_element_type=jnp.float32,
    ) * inv_l

    packed = packed_ref[...]
    tb = packed.shape[0]
    z = jnp.zeros((tb, b_ref.shape[1]), jnp.float32) + b_ref[...]
    for p in range(4):
        plane = ((packed >> (8 * p)) & 255).astype(jnp.float32)
        z = z + lax.dot_general(
            plane, m_ref[pl.ds(256 * p, 256), :],
            dimension_numbers=(((1,), (0,)), ((), ())),
            preferred_element_type=jnp.float32,
        )
    zmax = jnp.max(z, axis=1, keepdims=True)
    s = z - zmax
    lse = jnp.log(jnp.sum(jnp.exp(s), axis=1, keepdims=True))
    out_ref[...] = s - lse


def kernel(inputs, emb, W, b):
    B, L = inputs.shape
    V, D = emb.shape
    C = W.shape[0]

    packed = _sc_hist(inputs, b=B, hist=L).reshape(B, WORDS)

    TB = 512
    b2 = b.reshape(1, C)
    out = pl.pallas_call(
        functools.partial(_final_kernel, inv_l=1.0 / L, vocab=V),
        grid=(B // TB,),
        in_specs=[
            pl.BlockSpec((TB, WORDS), lambda i: (i, 0)),
            pl.BlockSpec((V, D), lambda i: (0, 0)),
            pl.BlockSpec((C, D), lambda i: (0, 0)),
            pl.BlockSpec((1, C), lambda i: (0, 0)),
        ],
        out_specs=pl.BlockSpec((TB, C), lambda i: (i, 0)),
        out_shape=jax.ShapeDtypeStruct((B, C), jnp.float32),
        scratch_shapes=[pltpu.VMEM((VOCAB_PAD, C), jnp.float32)],
    )(packed, emb, W, b2)

    return out


# speedup vs baseline: 71.5259x; 1.2007x over previous
import functools
import jax
import jax.numpy as jnp
from jax import lax
from jax.experimental import pallas as pl
from jax.experimental.pallas import tpu as pltpu
from jax.experimental.pallas import tpu_sc as plsc


VOCAB_PAD = 1024
WORDS = VOCAB_PAD // 4
LANES = 16
NC, NS = 2, 16
NW = NC * NS


def _sc_hist_body(idx_hbm, out_hbm, idx_v, cnt_v, sem, *, rows_per_w, hist):
    wid = lax.axis_index("s") * NC + lax.axis_index("c")
    base = wid * rows_per_w

    stage = pltpu.async_copy(idx_hbm.at[pl.ds(base, rows_per_w)], idx_v, sem)

    zero16 = jnp.zeros((LANES,), jnp.int32)
    groups_per_row = WORDS // LANES

    @plsc.parallel_loop(0, rows_per_w, unroll=2)
    def _(r):
        for j in range(groups_per_row):
            cnt_v[r, pl.ds(j * LANES, LANES)] = zero16

    stage.wait()

    nfull = hist // LANES
    tail = hist - nfull * LANES
    one = jnp.full((LANES,), 1, jnp.int32)
    lane = lax.iota(jnp.int32, LANES)

    def scatter_group(tile_base, vidx, mask):
        w = vidx & 255
        flat = tile_base + w
        val = one << ((vidx >> 8) << 3)
        plsc.addupdate_scatter(cnt_v, [flat >> 8, flat & 255], val, mask=mask)

    @plsc.parallel_loop(0, rows_per_w, unroll=2)
    def _(r):
        tile_base = r * WORDS
        for j in range(nfull):
            vidx = idx_v[r, pl.ds(j * LANES, LANES)]
            scatter_group(tile_base, vidx, None)
        if tail:
            vidx = idx_v[r, pl.ds(hist - LANES, LANES)]
            scatter_group(tile_base, vidx, lane >= (LANES - tail))

    pltpu.sync_copy(cnt_v, out_hbm.at[pl.ds(base, rows_per_w), :])


def _sc_hist(inputs, *, b, hist):
    rows_per_w = b // NW
    mesh = plsc.VectorSubcoreMesh(core_axis_name="c", subcore_axis_name="s")
    body = functools.partial(_sc_hist_body, rows_per_w=rows_per_w, hist=hist)
    f = pl.kernel(
        body,
        out_type=jax.ShapeDtypeStruct((b, WORDS), jnp.int32),
        mesh=mesh,
        scratch_types=[
            pltpu.VMEM((rows_per_w, hist), jnp.int32),
            pltpu.VMEM((rows_per_w, WORDS), jnp.int32),
            pltpu.SemaphoreType.DMA,
        ],
        compiler_params=pltpu.CompilerParams(needs_layout_passes=False),
    )
    return f(inputs)


def _final_kernel(packed_ref, emb_ref, w_ref, b_ref, out_ref, m_ref, *,
                  inv_l, vocab):
    @pl.when(pl.program_id(0) == 0)
    def _():
        m_ref[...] = jnp.zeros_like(m_ref)
        m_ref[:vocab, :] = lax.dot_general(
            emb_ref[...], w_ref[...],
            dimension_numbers=(((1,), (1,)), ((), ())),
            preferred_element_type=jnp.float32,
        ) * inv_l

    packed = packed_ref[...]
    tb = packed.shape[0]
    z = jnp.zeros((tb, b_ref.shape[1]), jnp.float32) + b_ref[...]
    for p in range(4):
        plane = ((packed >> (8 * p)) & 255).astype(jnp.float32)
        z = z + lax.dot_general(
            plane, m_ref[pl.ds(256 * p, 256), :],
            dimension_numbers=(((1,), (0,)), ((), ())),
            preferred_element_type=jnp.float32,
        )
    zmax = jnp.max(z, axis=1, keepdims=True)
    s = z - zmax
    lse = jnp.log(jnp.sum(jnp.exp(s), axis=1, keepdims=True))
    out_ref[...] = s - lse


def kernel(inputs, emb, W, b):
    B, L = inputs.shape
    V, D = emb.shape
    C = W.shape[0]

    packed = _sc_hist(inputs, b=B, hist=L)

    TB = 1024
    b2 = b.reshape(1, C)
    out = pl.pallas_call(
        functools.partial(_final_kernel, inv_l=1.0 / L, vocab=V),
        grid=(B // TB,),
        in_specs=[
            pl.BlockSpec((TB, WORDS), lambda i: (i, 0)),
            pl.BlockSpec((V, D), lambda i: (0, 0)),
            pl.BlockSpec((C, D), lambda i: (0, 0)),
            pl.BlockSpec((1, C), lambda i: (0, 0)),
        ],
        out_specs=pl.BlockSpec((TB, C), lambda i: (i, 0)),
        out_shape=jax.ShapeDtypeStruct((B, C), jnp.float32),
        scratch_shapes=[pltpu.VMEM((VOCAB_PAD, C), jnp.float32)],
    )(packed, emb, W, b2)

    return out
